# Initial kernel scaffold; baseline (speedup 1.0000x reference)
import jax
import jax.numpy as jnp
from jax.experimental import pallas as pl


def kernel(in_tensor, conditioning_code, tables):
    raise NotImplementedError("write your pallas kernel here")


# speedup vs baseline: 2.0128x; 2.0128x over previous
import functools
import numpy as np
import jax
import jax.numpy as jnp
from jax import lax
from jax.experimental import pallas as pl
from jax.experimental.pallas import tpu as pltpu, tpu_sc as plsc

_N_LEVELS = 16
_F = 2
_T = 2 ** 19
_BASE_RES = 16
_SCALE = 1.4472692012786865
_N_HASH = 4
_B = 131072
_P1 = int(np.int32(np.uint32(2654435761)))
_P2 = int(np.int32(np.uint32(805459861)))
_RES = [int(np.floor(_BASE_RES * _SCALE ** l)) for l in range(_N_LEVELS)]
_DENSE = [(r + 1) ** 3 <= _T for r in _RES]

_NC, _NS, _LANES = 2, 16, 16
_NW = _NC * _NS
_PTS_W = _B // _NW
_BATCH = 16
_CHUNK = 64
_NCHUNK = _PTS_W // (_BATCH * _CHUNK)
_OUTCOLS = _BATCH * _CHUNK


def _tec_body(x_hbm, c_hbm, tab_hbm, out_hbm, xbuf, cbuf, idxbuf, rowsbuf,
              outbuf, sem):
    wid = lax.axis_index("c") * _NS + lax.axis_index("s")
    base = wid * _PTS_W

    pltpu.sync_copy(x_hbm.at[:, pl.ds(base, _PTS_W)], xbuf)
    pltpu.sync_copy(c_hbm.at[:, pl.ds(base, _PTS_W)], cbuf)

    iota = jnp.arange(_BATCH, dtype=jnp.int32)

    @pl.loop(0, _NCHUNK)
    def chunk_loop(ch):
        @pl.loop(0, _CHUNK)
        def batch_loop(gi):
            off = ch * _OUTCOLS + gi * _BATCH
            x0 = xbuf[0, pl.ds(off, _BATCH)]
            x1 = xbuf[1, pl.ds(off, _BATCH)]
            x2 = xbuf[2, pl.ds(off, _BATCH)]

            copies = []
            for l in range(_N_LEVELS):
                res = _RES[l]
                p0 = (x0 * res).astype(jnp.int32)
                p1 = (x1 * res).astype(jnp.int32)
                p2 = (x2 * res).astype(jnp.int32)
                lbase = l * _T
                if _DENSE[l]:
                    s = res + 1
                    bi = p0 + s * p1 + (s * s) * p2 + lbase
                    for corner in range(8):
                        b0, b1, b2 = corner & 1, (corner >> 1) & 1, (corner >> 2) & 1
                        row = bi + (b0 + s * b1 + s * s * b2)
                        idxbuf[l, pl.ds(corner * _BATCH, _BATCH)] = row
                else:
                    h1a = p1 * _P1
                    h1b = (p1 + 1) * _P1
                    h2a = p2 * _P2
                    h2b = (p2 + 1) * _P2
                    p0b = p0 + 1
                    for corner in range(8):
                        a = p0b if (corner & 1) else p0
                        b = h1b if (corner & 2) else h1a
                        c = h2b if (corner & 4) else h2a
                        row = ((a ^ b ^ c) & (_T - 1)) + lbase
                        idxbuf[l, pl.ds(corner * _BATCH, _BATCH)] = row
                copies.append(
                    pltpu.async_copy(tab_hbm.at[idxbuf.at[l]], rowsbuf.at[l], sem))
            for cp in copies:
                cp.wait()

            c0 = cbuf[0, pl.ds(off, _BATCH)]
            c1 = cbuf[1, pl.ds(off, _BATCH)]
            c2 = cbuf[2, pl.ds(off, _BATCH)]
            c3 = cbuf[3, pl.ds(off, _BATCH)]
            ocol = gi * _BATCH
            for l in range(_N_LEVELS):
                res = _RES[l]
                pf0 = x0 * res
                pf1 = x1 * res
                pf2 = x2 * res
                w0 = pf0 - pf0.astype(jnp.int32).astype(jnp.float32)
                w1 = pf1 - pf1.astype(jnp.int32).astype(jnp.float32)
                w2 = pf2 - pf2.astype(jnp.int32).astype(jnp.float32)
                u0 = 1.0 - w0
                u1 = 1.0 - w1
                u2 = 1.0 - w2
                m = [u1 * u2, w1 * u2, u1 * w2, w1 * w2]
                acc0 = jnp.zeros((_BATCH,), jnp.float32)
                acc1 = jnp.zeros((_BATCH,), jnp.float32)
                for corner in range(8):
                    wc = (w0 if (corner & 1) else u0) * m[corner >> 1]
                    rv = iota + corner * _BATCH
                    lv = jnp.full((_BATCH,), l, jnp.int32)
                    g = [plsc.load_gather(rowsbuf,
                                          [lv, rv, jnp.full((_BATCH,), j, jnp.int32)])
                         for j in range(8)]
                    f0 = g[0] * c0 + g[2] * c1 + g[4] * c2 + g[6] * c3
                    f1 = g[1] * c0 + g[3] * c1 + g[5] * c2 + g[7] * c3
                    acc0 = acc0 + wc * f0
                    acc1 = acc1 + wc * f1
                outbuf[2 * l, pl.ds(ocol, _BATCH)] = acc0
                outbuf[2 * l + 1, pl.ds(ocol, _BATCH)] = acc1

        pltpu.sync_copy(outbuf, out_hbm.at[:, pl.ds(base + ch * _OUTCOLS, _OUTCOLS)])


def kernel(in_tensor, conditioning_code, tables):
    xT = in_tensor.T
    cT = conditioning_code.T
    tab = jnp.transpose(tables, (1, 2, 0, 3)).reshape(_N_LEVELS * _T,
                                                      _N_HASH * _F)
    mesh = plsc.VectorSubcoreMesh(core_axis_name="c", subcore_axis_name="s")
    out = pl.kernel(
        _tec_body,
        out_type=jax.ShapeDtypeStruct((2 * _N_LEVELS, _B), jnp.float32),
        mesh=mesh,
        scratch_types=[
            pltpu.VMEM((3, _PTS_W), jnp.float32),
            pltpu.VMEM((4, _PTS_W), jnp.float32),
            pltpu.VMEM((_N_LEVELS, 8 * _BATCH), jnp.int32),
            pltpu.VMEM((_N_LEVELS, 8 * _BATCH, _N_HASH * _F), jnp.float32),
            pltpu.VMEM((2 * _N_LEVELS, _OUTCOLS), jnp.float32),
            pltpu.SemaphoreType.DMA,
        ],
        compiler_params=pltpu.CompilerParams(needs_layout_passes=False,
                                             use_tc_tiling_on_sc=False),
    )(xT, cT, tab)
    return out.T
